# baseline (speedup 1.0000x reference)
import numpy as np
import jax
import jax.numpy as jnp
from jax.experimental import pallas as pl
from jax.experimental.pallas import tpu as pltpu

D = 48
H = 2
HD = D // H
N = 21
ND = N * D
ITERS = 3
EPS = 1e-5

_SEG = np.kron(np.eye(N, dtype=np.float32), np.ones((D, 1), np.float32))


def _body(kv_ref, g0a_ref, g0b_ref, mta_ref, mtb_ref, vta_ref, vtb_ref,
          pva_ref, pvb_ref, s_ref, st_ref, oc_ref, wih_ref, bih_ref,
          whh_ref, bhh_ref, gh0_ref, q0_ref, isig_ref, p_ref, nip_ref,
          p1_ref, b1_ref, p2_ref, b2_ref, fused_ref, aww_ref):
    f32 = jnp.float32
    kv = kv_ref[...]
    S = s_ref[...]
    ST = st_ref[...]

    ssum = jnp.dot(kv, S, preferred_element_type=f32, precision=jax.lax.Precision.HIGHEST)
    ssq = jnp.dot(kv * kv, S, preferred_element_type=f32, precision=jax.lax.Precision.HIGHEST)
    mu = ssum * (1.0 / D)
    var = ssq * (1.0 / D) - mu * mu
    rstd = jax.lax.rsqrt(var + EPS)
    z = (kv - jnp.dot(mu, ST, preferred_element_type=f32, precision=jax.lax.Precision.HIGHEST)) \
        * jnp.dot(rstd, ST, preferred_element_type=f32, precision=jax.lax.Precision.HIGHEST)

    def softmax_n(s):
        m = jnp.max(s, axis=-1, keepdims=True)
        e = jnp.exp(s - m)
        return e / jnp.sum(e, axis=-1, keepdims=True)

    def attn_out(ga, gb):
        s0 = jnp.dot(z * ga, S, preferred_element_type=f32, precision=jax.lax.Precision.HIGHEST)
        s1 = jnp.dot(z * gb, S, preferred_element_type=f32, precision=jax.lax.Precision.HIGHEST)
        aw0 = softmax_n(s0)
        aw1 = softmax_n(s1)
        e0 = jnp.dot(aw0, ST, preferred_element_type=f32, precision=jax.lax.Precision.HIGHEST)
        e1 = jnp.dot(aw1, ST, preferred_element_type=f32, precision=jax.lax.Precision.HIGHEST)
        out = (jnp.dot(z * e0, pva_ref[...], preferred_element_type=f32, precision=jax.lax.Precision.HIGHEST)
               + jnp.dot(z * e1, pvb_ref[...], preferred_element_type=f32, precision=jax.lax.Precision.HIGHEST)
               + oc_ref[...])
        return out, aw0, aw1

    def gru(out, gh, q):
        gi = jnp.dot(out, wih_ref[...], preferred_element_type=f32, precision=jax.lax.Precision.HIGHEST) \
            + bih_ref[...]
        r = jax.nn.sigmoid(gi[:, :D] + gh[:, :D])
        zg = jax.nn.sigmoid(gi[:, D:2 * D] + gh[:, D:2 * D])
        n = jnp.tanh(gi[:, 2 * D:] + r * gh[:, 2 * D:])
        return (1.0 - zg) * n + zg * q

    out, aw0, aw1 = attn_out(g0a_ref[...], g0b_ref[...])
    q = gru(out, gh0_ref[...], q0_ref[...])

    for _ in range(ITERS - 1):
        qmu = jnp.mean(q, axis=-1, keepdims=True)
        qc = q - qmu
        qvar = jnp.mean(qc * qc, axis=-1, keepdims=True)
        zq = qc * jax.lax.rsqrt(qvar + EPS)
        ga = jnp.dot(zq, mta_ref[...], preferred_element_type=f32, precision=jax.lax.Precision.HIGHEST) \
            + vta_ref[...]
        gb = jnp.dot(zq, mtb_ref[...], preferred_element_type=f32, precision=jax.lax.Precision.HIGHEST) \
            + vtb_ref[...]
        out, aw0, aw1 = attn_out(ga, gb)
        gh = jnp.dot(q, whh_ref[...], preferred_element_type=f32, precision=jax.lax.Precision.HIGHEST) \
            + bhh_ref[...]
        q = gru(out, gh, q)

    aww_ref[...] = (aw0 + aw1) * 0.5

    ratio = jnp.minimum(jnp.abs(q) * isig_ref[...], 15.0)
    rp = jnp.exp2(p_ref[...] * jnp.log2(jnp.maximum(ratio, 1e-30)))
    f = q * jnp.exp2(nip_ref[...] * jnp.log2(1.0 + rp))

    f = jnp.maximum(
        jnp.dot(f, p1_ref[...], preferred_element_type=f32, precision=jax.lax.Precision.HIGHEST) + b1_ref[...],
        0.0)
    fused_ref[...] = jnp.dot(f, p2_ref[...], preferred_element_type=f32, precision=jax.lax.Precision.HIGHEST) \
        + b2_ref[...]


def kernel(slot_outputs, fusion_query, in_proj_w, in_proj_b, out_proj_w,
           out_proj_b, ln_q_g, ln_q_b, ln_kv_g, ln_kv_b, gru_w_ih,
           gru_w_hh, gru_b_ih, gru_b_hh, sigma_y_raw, p_raw,
           proj1_w, proj1_b, proj2_w, proj2_b):
    B, T, _, _ = slot_outputs.shape
    M = B * T
    f32 = jnp.float32

    wq, wk, wv = in_proj_w[:D], in_proj_w[D:2 * D], in_proj_w[2 * D:]
    bq = in_proj_b[:D]
    bv = in_proj_b[2 * D:]
    WoT = out_proj_w.T
    scale = 1.0 / np.sqrt(HD)

    def head_mats(h):
        sl = slice(h * HD, (h + 1) * HD)
        wqh, wkh, wvh = wq[sl], wk[sl], wv[sl]
        Mh = scale * (ln_q_g[:, None] * (wqh.T @ wkh)) * ln_kv_g[None, :]
        vh = scale * (((wqh @ ln_q_b + bq[sl]) @ wkh) * ln_kv_g)
        Ph = (ln_kv_g[:, None] * wvh.T) @ WoT[sl]
        return Mh, vh, Ph

    M0, v0, P0 = head_mats(0)
    M1, v1, P1 = head_mats(1)
    Mta = jnp.tile(M0, (1, N))
    Mtb = jnp.tile(M1, (1, N))
    vta = jnp.tile(v0, N)[None]
    vtb = jnp.tile(v1, N)[None]
    Pva = jnp.tile(P0, (N, 1))
    Pvb = jnp.tile(P1, (N, 1))
    out_const = (out_proj_b + (ln_kv_b @ wv.T + bv) @ WoT)[None]

    fq = fusion_query
    mu0 = fq.mean()
    c0 = fq - mu0
    zq0 = c0 * jax.lax.rsqrt((c0 * c0).mean() + EPS)
    g0a = (zq0 @ Mta + vta)
    g0b = (zq0 @ Mtb + vtb)
    gh0 = (fq @ gru_w_hh.T + gru_b_hh)[None]
    q0 = fq[None]

    sigma_y = jax.nn.softplus(sigma_y_raw) + 0.01
    isig = (1.0 / sigma_y)[None]
    p = 1.5 + jax.nn.softplus(p_raw)
    p_arr = p[:, None]
    nip = (-1.0 / p)[:, None]

    kv2 = slot_outputs.reshape(M, ND)
    R = 512
    while M % R:
        R //= 2
    grid = (M // R,)

    def const(shape):
        return pl.BlockSpec(shape, lambda i: (0, 0))

    fused, aww = pl.pallas_call(
        _body,
        grid=grid,
        in_specs=[
            pl.BlockSpec((R, ND), lambda i: (i, 0)),
            const((1, ND)), const((1, ND)),
            const((D, ND)), const((D, ND)),
            const((1, ND)), const((1, ND)),
            const((ND, D)), const((ND, D)),
            const((ND, N)), const((N, ND)),
            const((1, D)),
            const((D, 3 * D)), const((1, 3 * D)),
            const((D, 3 * D)), const((1, 3 * D)),
            const((1, 3 * D)), const((1, D)),
            const((1, D)), const((1, 1)), const((1, 1)),
            const((D, D)), const((1, D)), const((D, D)), const((1, D)),
        ],
        out_specs=[
            pl.BlockSpec((R, D), lambda i: (i, 0)),
            pl.BlockSpec((R, N), lambda i: (i, 0)),
        ],
        out_shape=[
            jax.ShapeDtypeStruct((M, D), f32),
            jax.ShapeDtypeStruct((M, N), f32),
        ],
        compiler_params=pltpu.CompilerParams(
            dimension_semantics=("parallel",),
            vmem_limit_bytes=48 * 1024 * 1024),
    )(kv2, g0a, g0b, Mta, Mtb, vta, vtb, Pva, Pvb,
      jnp.asarray(_SEG), jnp.asarray(_SEG.T), out_const,
      gru_w_ih.T, gru_b_ih[None], gru_w_hh.T, gru_b_hh[None], gh0, q0,
      isig, p_arr, nip,
      proj1_w.T, proj1_b[None], proj2_w.T, proj2_b[None])

    return fused.reshape(B, T, D), aww.reshape(B, T, N)


# speedup vs baseline: 3.7267x; 3.7267x over previous
import numpy as np
import jax
import jax.numpy as jnp
from jax.experimental import pallas as pl
from jax.experimental.pallas import tpu as pltpu

D = 48
H = 2
HD = D // H
N = 21
ND = N * D
ITERS = 3
EPS = 1e-5

_SEG = np.kron(np.eye(N, dtype=np.float32), np.ones((D, 1), np.float32))


def _body(kv_ref, g0a_ref, g0b_ref, mta_ref, mtb_ref, vta_ref, vtb_ref,
          pva_ref, pvb_ref, s_ref, st_ref, oc_ref, wih_ref, bih_ref,
          whh_ref, bhh_ref, gh0_ref, q0_ref, isig_ref, p_ref, nip_ref,
          p1_ref, b1_ref, p2_ref, b2_ref, fused_ref, aww_ref):
    f32 = jnp.float32
    kv = kv_ref[...]
    S = s_ref[...]
    ST = st_ref[...]

    ssum = jnp.dot(kv, S, preferred_element_type=f32)
    ssq = jnp.dot(kv * kv, S, preferred_element_type=f32)
    mu = ssum * (1.0 / D)
    var = ssq * (1.0 / D) - mu * mu
    rstd = jax.lax.rsqrt(var + EPS)
    z = (kv - jnp.dot(mu, ST, preferred_element_type=f32)) \
        * jnp.dot(rstd, ST, preferred_element_type=f32)

    def softmax_n(s):
        m = jnp.max(s, axis=-1, keepdims=True)
        e = jnp.exp(s - m)
        return e / jnp.sum(e, axis=-1, keepdims=True)

    def attn_out(ga, gb):
        s0 = jnp.dot(z * ga, S, preferred_element_type=f32)
        s1 = jnp.dot(z * gb, S, preferred_element_type=f32)
        aw0 = softmax_n(s0)
        aw1 = softmax_n(s1)
        e0 = jnp.dot(aw0, ST, preferred_element_type=f32)
        e1 = jnp.dot(aw1, ST, preferred_element_type=f32)
        out = (jnp.dot(z * e0, pva_ref[...], preferred_element_type=f32)
               + jnp.dot(z * e1, pvb_ref[...], preferred_element_type=f32)
               + oc_ref[...])
        return out, aw0, aw1

    def gru(out, gh, q):
        gi = jnp.dot(out, wih_ref[...], preferred_element_type=f32) \
            + bih_ref[...]
        r = jax.nn.sigmoid(gi[:, :D] + gh[:, :D])
        zg = jax.nn.sigmoid(gi[:, D:2 * D] + gh[:, D:2 * D])
        n = jnp.tanh(gi[:, 2 * D:] + r * gh[:, 2 * D:])
        return (1.0 - zg) * n + zg * q

    out, aw0, aw1 = attn_out(g0a_ref[...], g0b_ref[...])
    q = gru(out, gh0_ref[...], q0_ref[...])

    for _ in range(ITERS - 1):
        qmu = jnp.mean(q, axis=-1, keepdims=True)
        qc = q - qmu
        qvar = jnp.mean(qc * qc, axis=-1, keepdims=True)
        zq = qc * jax.lax.rsqrt(qvar + EPS)
        ga = jnp.dot(zq, mta_ref[...], preferred_element_type=f32) \
            + vta_ref[...]
        gb = jnp.dot(zq, mtb_ref[...], preferred_element_type=f32) \
            + vtb_ref[...]
        out, aw0, aw1 = attn_out(ga, gb)
        gh = jnp.dot(q, whh_ref[...], preferred_element_type=f32) \
            + bhh_ref[...]
        q = gru(out, gh, q)

    aww_ref[...] = (aw0 + aw1) * 0.5

    ratio = jnp.minimum(jnp.abs(q) * isig_ref[...], 15.0)
    rp = jnp.exp2(p_ref[...] * jnp.log2(jnp.maximum(ratio, 1e-30)))
    f = q * jnp.exp2(nip_ref[...] * jnp.log2(1.0 + rp))

    f = jnp.maximum(
        jnp.dot(f, p1_ref[...], preferred_element_type=f32) + b1_ref[...],
        0.0)
    fused_ref[...] = jnp.dot(f, p2_ref[...], preferred_element_type=f32) \
        + b2_ref[...]


def kernel(slot_outputs, fusion_query, in_proj_w, in_proj_b, out_proj_w,
           out_proj_b, ln_q_g, ln_q_b, ln_kv_g, ln_kv_b, gru_w_ih,
           gru_w_hh, gru_b_ih, gru_b_hh, sigma_y_raw, p_raw,
           proj1_w, proj1_b, proj2_w, proj2_b):
    B, T, _, _ = slot_outputs.shape
    M = B * T
    f32 = jnp.float32

    wq, wk, wv = in_proj_w[:D], in_proj_w[D:2 * D], in_proj_w[2 * D:]
    bq = in_proj_b[:D]
    bv = in_proj_b[2 * D:]
    WoT = out_proj_w.T
    scale = 1.0 / np.sqrt(HD)

    def head_mats(h):
        sl = slice(h * HD, (h + 1) * HD)
        wqh, wkh, wvh = wq[sl], wk[sl], wv[sl]
        Mh = scale * (ln_q_g[:, None] * (wqh.T @ wkh)) * ln_kv_g[None, :]
        vh = scale * (((wqh @ ln_q_b + bq[sl]) @ wkh) * ln_kv_g)
        Ph = (ln_kv_g[:, None] * wvh.T) @ WoT[sl]
        return Mh, vh, Ph

    M0, v0, P0 = head_mats(0)
    M1, v1, P1 = head_mats(1)
    Mta = jnp.tile(M0, (1, N))
    Mtb = jnp.tile(M1, (1, N))
    vta = jnp.tile(v0, N)[None]
    vtb = jnp.tile(v1, N)[None]
    Pva = jnp.tile(P0, (N, 1))
    Pvb = jnp.tile(P1, (N, 1))
    out_const = (out_proj_b + (ln_kv_b @ wv.T + bv) @ WoT)[None]

    fq = fusion_query
    mu0 = fq.mean()
    c0 = fq - mu0
    zq0 = c0 * jax.lax.rsqrt((c0 * c0).mean() + EPS)
    g0a = (zq0 @ Mta + vta)
    g0b = (zq0 @ Mtb + vtb)
    gh0 = (fq @ gru_w_hh.T + gru_b_hh)[None]
    q0 = fq[None]

    sigma_y = jax.nn.softplus(sigma_y_raw) + 0.01
    isig = (1.0 / sigma_y)[None]
    p = 1.5 + jax.nn.softplus(p_raw)
    p_arr = p[:, None]
    nip = (-1.0 / p)[:, None]

    kv2 = slot_outputs.reshape(M, ND)
    R = 512
    while M % R:
        R //= 2
    grid = (M // R,)

    def const(shape):
        return pl.BlockSpec(shape, lambda i: (0, 0))

    fused, aww = pl.pallas_call(
        _body,
        grid=grid,
        in_specs=[
            pl.BlockSpec((R, ND), lambda i: (i, 0)),
            const((1, ND)), const((1, ND)),
            const((D, ND)), const((D, ND)),
            const((1, ND)), const((1, ND)),
            const((ND, D)), const((ND, D)),
            const((ND, N)), const((N, ND)),
            const((1, D)),
            const((D, 3 * D)), const((1, 3 * D)),
            const((D, 3 * D)), const((1, 3 * D)),
            const((1, 3 * D)), const((1, D)),
            const((1, D)), const((1, 1)), const((1, 1)),
            const((D, D)), const((1, D)), const((D, D)), const((1, D)),
        ],
        out_specs=[
            pl.BlockSpec((R, D), lambda i: (i, 0)),
            pl.BlockSpec((R, N), lambda i: (i, 0)),
        ],
        out_shape=[
            jax.ShapeDtypeStruct((M, D), f32),
            jax.ShapeDtypeStruct((M, N), f32),
        ],
        compiler_params=pltpu.CompilerParams(
            dimension_semantics=("parallel",),
            vmem_limit_bytes=48 * 1024 * 1024),
    )(kv2, g0a, g0b, Mta, Mtb, vta, vtb, Pva, Pvb,
      jnp.asarray(_SEG), jnp.asarray(_SEG.T), out_const,
      gru_w_ih.T, gru_b_ih[None], gru_w_hh.T, gru_b_hh[None], gh0, q0,
      isig, p_arr, nip,
      proj1_w.T, proj1_b[None], proj2_w.T, proj2_b[None])

    return fused.reshape(B, T, D), aww.reshape(B, T, N)


# speedup vs baseline: 4.0147x; 1.0773x over previous
import numpy as np
import jax
import jax.numpy as jnp
from jax.experimental import pallas as pl
from jax.experimental.pallas import tpu as pltpu

D = 48
H = 2
HD = D // H
N = 21
ND = N * D
ITERS = 3
EPS = 1e-5

_SEG = np.kron(np.eye(N, dtype=np.float32), np.ones((D, 1), np.float32))


def _body(kv_ref, g0a_ref, g0b_ref, mta_ref, mtb_ref, vta_ref, vtb_ref,
          pva_ref, pvb_ref, s_ref, st_ref, oc_ref, wih_ref, bih_ref,
          whh_ref, bhh_ref, gh0_ref, q0_ref, isig_ref, p_ref, nip_ref,
          p1_ref, b1_ref, p2_ref, b2_ref, fused_ref, aww_ref):
    f32 = jnp.float32
    kv = kv_ref[...]
    S = s_ref[...]
    ST = st_ref[...]

    ssum = jnp.dot(kv, S, preferred_element_type=f32)
    ssq = jnp.dot(kv * kv, S, preferred_element_type=f32)
    mu = ssum * (1.0 / D)
    var = ssq * (1.0 / D) - mu * mu
    rstd = jax.lax.rsqrt(var + EPS)
    z = (kv - jnp.dot(mu, ST, preferred_element_type=f32)) \
        * jnp.dot(rstd, ST, preferred_element_type=f32)

    def softmax_n(s):
        m = jnp.max(s, axis=-1, keepdims=True)
        e = jnp.exp(s - m)
        return e / jnp.sum(e, axis=-1, keepdims=True)

    def attn_out(ga, gb):
        s0 = jnp.dot(z * ga, S, preferred_element_type=f32)
        s1 = jnp.dot(z * gb, S, preferred_element_type=f32)
        aw0 = softmax_n(s0)
        aw1 = softmax_n(s1)
        e0 = jnp.dot(aw0, ST, preferred_element_type=f32)
        e1 = jnp.dot(aw1, ST, preferred_element_type=f32)
        out = (jnp.dot(z * e0, pva_ref[...], preferred_element_type=f32)
               + jnp.dot(z * e1, pvb_ref[...], preferred_element_type=f32)
               + oc_ref[...])
        return out, aw0, aw1

    def gru(out, gh, q):
        gi = jnp.dot(out, wih_ref[...], preferred_element_type=f32) \
            + bih_ref[...]
        r = jax.nn.sigmoid(gi[:, :D] + gh[:, :D])
        zg = jax.nn.sigmoid(gi[:, D:2 * D] + gh[:, D:2 * D])
        n = jnp.tanh(gi[:, 2 * D:] + r * gh[:, 2 * D:])
        return (1.0 - zg) * n + zg * q

    out, aw0, aw1 = attn_out(g0a_ref[...], g0b_ref[...])
    q = gru(out, gh0_ref[...], q0_ref[...])

    for _ in range(ITERS - 1):
        qmu = jnp.mean(q, axis=-1, keepdims=True)
        qc = q - qmu
        qvar = jnp.mean(qc * qc, axis=-1, keepdims=True)
        zq = qc * jax.lax.rsqrt(qvar + EPS)
        ga = jnp.dot(zq, mta_ref[...], preferred_element_type=f32) \
            + vta_ref[...]
        gb = jnp.dot(zq, mtb_ref[...], preferred_element_type=f32) \
            + vtb_ref[...]
        out, aw0, aw1 = attn_out(ga, gb)
        gh = jnp.dot(q, whh_ref[...], preferred_element_type=f32) \
            + bhh_ref[...]
        q = gru(out, gh, q)

    aww_ref[...] = (aw0 + aw1) * 0.5

    ratio = jnp.minimum(jnp.abs(q) * isig_ref[...], 15.0)
    rp = jnp.exp2(p_ref[...] * jnp.log2(jnp.maximum(ratio, 1e-30)))
    f = q * jnp.exp2(nip_ref[...] * jnp.log2(1.0 + rp))

    f = jnp.maximum(
        jnp.dot(f, p1_ref[...], preferred_element_type=f32) + b1_ref[...],
        0.0)
    fused_ref[...] = jnp.dot(f, p2_ref[...], preferred_element_type=f32) \
        + b2_ref[...]


def kernel(slot_outputs, fusion_query, in_proj_w, in_proj_b, out_proj_w,
           out_proj_b, ln_q_g, ln_q_b, ln_kv_g, ln_kv_b, gru_w_ih,
           gru_w_hh, gru_b_ih, gru_b_hh, sigma_y_raw, p_raw,
           proj1_w, proj1_b, proj2_w, proj2_b):
    B, T, _, _ = slot_outputs.shape
    M = B * T
    f32 = jnp.float32

    wq, wk, wv = in_proj_w[:D], in_proj_w[D:2 * D], in_proj_w[2 * D:]
    bq = in_proj_b[:D]
    bv = in_proj_b[2 * D:]
    WoT = out_proj_w.T
    scale = 1.0 / np.sqrt(HD)

    def head_mats(h):
        sl = slice(h * HD, (h + 1) * HD)
        wqh, wkh, wvh = wq[sl], wk[sl], wv[sl]
        Mh = scale * (ln_q_g[:, None] * (wqh.T @ wkh)) * ln_kv_g[None, :]
        vh = scale * (((wqh @ ln_q_b + bq[sl]) @ wkh) * ln_kv_g)
        Ph = (ln_kv_g[:, None] * wvh.T) @ WoT[sl]
        return Mh, vh, Ph

    M0, v0, P0 = head_mats(0)
    M1, v1, P1 = head_mats(1)
    Mta = jnp.tile(M0, (1, N))
    Mtb = jnp.tile(M1, (1, N))
    vta = jnp.tile(v0, N)[None]
    vtb = jnp.tile(v1, N)[None]
    Pva = jnp.tile(P0, (N, 1))
    Pvb = jnp.tile(P1, (N, 1))
    out_const = (out_proj_b + (ln_kv_b @ wv.T + bv) @ WoT)[None]

    fq = fusion_query
    mu0 = fq.mean()
    c0 = fq - mu0
    zq0 = c0 * jax.lax.rsqrt((c0 * c0).mean() + EPS)
    g0a = (zq0 @ Mta + vta)
    g0b = (zq0 @ Mtb + vtb)
    gh0 = (fq @ gru_w_hh.T + gru_b_hh)[None]
    q0 = fq[None]

    sigma_y = jax.nn.softplus(sigma_y_raw) + 0.01
    isig = (1.0 / sigma_y)[None]
    p = 1.5 + jax.nn.softplus(p_raw)
    p_arr = p[:, None]
    nip = (-1.0 / p)[:, None]

    kv2 = slot_outputs.reshape(M, ND)
    R = 1024
    while M % R:
        R //= 2
    grid = (M // R,)

    def const(shape):
        return pl.BlockSpec(shape, lambda i: (0, 0))

    fused, aww = pl.pallas_call(
        _body,
        grid=grid,
        in_specs=[
            pl.BlockSpec((R, ND), lambda i: (i, 0)),
            const((1, ND)), const((1, ND)),
            const((D, ND)), const((D, ND)),
            const((1, ND)), const((1, ND)),
            const((ND, D)), const((ND, D)),
            const((ND, N)), const((N, ND)),
            const((1, D)),
            const((D, 3 * D)), const((1, 3 * D)),
            const((D, 3 * D)), const((1, 3 * D)),
            const((1, 3 * D)), const((1, D)),
            const((1, D)), const((1, 1)), const((1, 1)),
            const((D, D)), const((1, D)), const((D, D)), const((1, D)),
        ],
        out_specs=[
            pl.BlockSpec((R, D), lambda i: (i, 0)),
            pl.BlockSpec((R, N), lambda i: (i, 0)),
        ],
        out_shape=[
            jax.ShapeDtypeStruct((M, D), f32),
            jax.ShapeDtypeStruct((M, N), f32),
        ],
        compiler_params=pltpu.CompilerParams(
            dimension_semantics=("arbitrary",),
            vmem_limit_bytes=48 * 1024 * 1024),
    )(kv2, g0a, g0b, Mta, Mtb, vta, vtb, Pva, Pvb,
      jnp.asarray(_SEG), jnp.asarray(_SEG.T), out_const,
      gru_w_ih.T, gru_b_ih[None], gru_w_hh.T, gru_b_hh[None], gh0, q0,
      isig, p_arr, nip,
      proj1_w.T, proj1_b[None], proj2_w.T, proj2_b[None])

    return fused.reshape(B, T, D), aww.reshape(B, T, N)
